# baseline (speedup 1.0000x reference)
import functools

import jax
import jax.numpy as jnp
from jax import lax
from jax.experimental import pallas as pl
from jax.experimental.pallas import tpu as pltpu
from jax.experimental.pallas import tpu_sc as plsc

E = 16
TOPK = 2
BLK = 1024
H = 2048
N_TOTAL = 32768
N_SC = 1024
N_TC = N_TOTAL - N_SC
NW = 32
TPW = N_SC // NW
CH = 16


def _epilogue(lt):
    m1 = jnp.max(lt, axis=0, keepdims=True)
    ex = jnp.exp(lt - m1)
    s = jnp.sum(ex, axis=0, keepdims=True)
    probs = ex * (1.0 / s)

    rows = lax.broadcasted_iota(jnp.int32, lt.shape, 0)
    i1 = jnp.min(jnp.where(lt == m1, rows, E), axis=0, keepdims=True)
    masked = jnp.where(rows == i1, -jnp.inf, lt)
    m2 = jnp.max(masked, axis=0, keepdims=True)
    i2 = jnp.min(jnp.where(masked == m2, rows, E), axis=0, keepdims=True)

    e2 = jnp.exp(m2 - m1)
    tw1 = 1.0 / (1.0 + e2)
    topw = jnp.concatenate([tw1, e2 * tw1], axis=0)
    tope = jnp.concatenate([i1, i2], axis=0)
    return probs, topw, tope


def _router_block(hs_ref, w_ref, weights_ref, topw_ref, tope_ref):
    hs = hs_ref[...]
    w = w_ref[...]
    lt = lax.dot_general(
        w, hs, (((1,), (1,)), ((), ())), preferred_element_type=jnp.float32
    )
    probs, topw, tope = _epilogue(lt)
    weights_ref[...] = probs
    topw_ref[...] = topw
    tope_ref[...] = tope


def _bf16_round(x):
    u = lax.bitcast_convert_type(x, jnp.uint32)
    lsb = lax.shift_right_logical(u, jnp.uint32(16)) & jnp.uint32(1)
    u = (u + jnp.uint32(0x7FFF) + lsb) & jnp.uint32(0xFFFF0000)
    return lax.bitcast_convert_type(u, jnp.float32)


def _sc_logits_body(hs_hbm, wt_hbm, out_hbm, wt_v, hs_v, out_v):
    c = lax.axis_index("c")
    s = lax.axis_index("s")
    wid = s * 2 + c
    pltpu.sync_copy(wt_hbm, wt_v)

    def chunk(ci, carry):
        t0 = wid * TPW + ci * CH
        pltpu.sync_copy(
            hs_hbm.at[pl.ds((N_TC + t0) * H, CH * H)], hs_v
        )
        for t in range(CH):
            def hb(h16, acc):
                hv = hs_v[pl.ds(t * H + h16 * 16, 16)]
                hv = _bf16_round(hv)
                for j in range(16):
                    acc = acc + hv[j] * wt_v[pl.ds((h16 * 16 + j) * E, E)]
                return acc
            acc = lax.fori_loop(0, H // 16, hb, jnp.zeros((E,), jnp.float32))
            out_v[pl.ds(t * E, E)] = acc
        pltpu.sync_copy(out_v, out_hbm.at[pl.ds(t0 * E, CH * E)])
        return carry

    lax.fori_loop(0, TPW // CH, chunk, 0)


def _sc_epilogue_block(lt_ref, weights_ref, topw_ref, tope_ref):
    probs, topw, tope = _epilogue(lt_ref[...])
    weights_ref[...] = probs
    topw_ref[...] = topw
    tope_ref[...] = tope


@jax.jit
def _router(hs2d, W):
    wt_tc, topw_tc, tope_tc = pl.pallas_call(
        _router_block,
        grid=(N_TC // BLK,),
        in_specs=[
            pl.BlockSpec((BLK, H), lambda i: (i, 0)),
            pl.BlockSpec((E, H), lambda i: (0, 0)),
        ],
        out_specs=[
            pl.BlockSpec((E, BLK), lambda i: (0, i)),
            pl.BlockSpec((TOPK, BLK), lambda i: (0, i)),
            pl.BlockSpec((TOPK, BLK), lambda i: (0, i)),
        ],
        out_shape=[
            jax.ShapeDtypeStruct((E, N_TC), jnp.float32),
            jax.ShapeDtypeStruct((TOPK, N_TC), jnp.float32),
            jax.ShapeDtypeStruct((TOPK, N_TC), jnp.int32),
        ],
    )(hs2d, W)

    sc_kernel = functools.partial(
        pl.kernel,
        mesh=plsc.VectorSubcoreMesh(core_axis_name="c", subcore_axis_name="s"),
        out_type=jax.ShapeDtypeStruct((N_SC * E,), jnp.float32),
        scratch_types=[
            pltpu.VMEM((H * E,), jnp.float32),
            pltpu.VMEM((CH * H,), jnp.float32),
            pltpu.VMEM((CH * E,), jnp.float32),
        ],
    )(_sc_logits_body)
    wt_rounded = _bf16_round(W.T)
    lt_sc = sc_kernel(hs2d.reshape(-1), wt_rounded.reshape(-1)).reshape(N_SC, E)

    wt_sc, topw_sc, tope_sc = pl.pallas_call(
        _sc_epilogue_block,
        grid=(1,),
        in_specs=[pl.BlockSpec((E, N_SC), lambda i: (0, 0))],
        out_specs=[
            pl.BlockSpec((E, N_SC), lambda i: (0, 0)),
            pl.BlockSpec((TOPK, N_SC), lambda i: (0, 0)),
            pl.BlockSpec((TOPK, N_SC), lambda i: (0, 0)),
        ],
        out_shape=[
            jax.ShapeDtypeStruct((E, N_SC), jnp.float32),
            jax.ShapeDtypeStruct((TOPK, N_SC), jnp.float32),
            jax.ShapeDtypeStruct((TOPK, N_SC), jnp.int32),
        ],
    )(lt_sc.T)

    weights = jnp.concatenate([wt_tc, wt_sc], axis=1).T
    top_weights = jnp.concatenate([topw_tc, topw_sc], axis=1).T
    top_experts = jnp.concatenate([tope_tc, tope_sc], axis=1).T
    return weights, top_weights, top_experts


def kernel(hidden_states, W):
    hs2d = hidden_states.reshape(-1, hidden_states.shape[-1])
    weights, top_weights, top_experts = _router(hs2d, W)
    weights = weights.astype(hidden_states.dtype)
    top_weights = top_weights.astype(hidden_states.dtype)
    return (weights, top_weights, top_experts)


# speedup vs baseline: 2.5150x; 2.5150x over previous
import functools

import jax
import jax.numpy as jnp
from jax import lax
from jax.experimental import pallas as pl
from jax.experimental.pallas import tpu as pltpu
from jax.experimental.pallas import tpu_sc as plsc

E = 16
TOPK = 2
BLK = 1024
H = 2048
N_TOTAL = 32768
N_SC = 1024
N_TC = N_TOTAL - N_SC
NW = 32
TPW = N_SC // NW
CH = 16


def _epilogue(lt):
    m1 = jnp.max(lt, axis=0, keepdims=True)
    ex = jnp.exp(lt - m1)
    s = jnp.sum(ex, axis=0, keepdims=True)
    probs = ex * (1.0 / s)

    rows = lax.broadcasted_iota(jnp.int32, lt.shape, 0)
    i1 = jnp.min(jnp.where(lt == m1, rows, E), axis=0, keepdims=True)
    masked = jnp.where(rows == i1, -jnp.inf, lt)
    m2 = jnp.max(masked, axis=0, keepdims=True)
    i2 = jnp.min(jnp.where(masked == m2, rows, E), axis=0, keepdims=True)

    e2 = jnp.exp(m2 - m1)
    tw1 = 1.0 / (1.0 + e2)
    topw = jnp.concatenate([tw1, e2 * tw1], axis=0)
    tope = jnp.concatenate([i1, i2], axis=0)
    return probs, topw, tope


def _router_block(hs_ref, w_ref, weights_ref, topw_ref, tope_ref):
    hs = hs_ref[...]
    w = w_ref[...]
    lt = lax.dot_general(
        w, hs, (((1,), (1,)), ((), ())), preferred_element_type=jnp.float32
    )
    probs, topw, tope = _epilogue(lt)
    weights_ref[...] = probs
    topw_ref[...] = topw
    tope_ref[...] = tope


def _bf16_round(x):
    u = lax.bitcast_convert_type(x, jnp.uint32)
    lsb = lax.shift_right_logical(u, jnp.uint32(16)) & jnp.uint32(1)
    u = (u + jnp.uint32(0x7FFF) + lsb) & jnp.uint32(0xFFFF0000)
    return lax.bitcast_convert_type(u, jnp.float32)


def _sc_logits_body(hs_hbm, wt_hbm, out_hbm, wt_v, hs_v, out_v):
    c = lax.axis_index("c")
    s = lax.axis_index("s")
    wid = s * 2 + c
    pltpu.sync_copy(wt_hbm, wt_v)

    def chunk(ci, carry):
        t0 = wid * TPW + ci * CH
        pltpu.sync_copy(hs_hbm.at[pl.ds(N_TC + t0, CH)], hs_v)
        for t in range(CH):
            def hb(h16, acc):
                hv = hs_v[t, pl.ds(h16 * 16, 16)]
                hv = _bf16_round(hv)
                for j in range(16):
                    acc = acc + hv[j] * wt_v[pl.ds((h16 * 16 + j) * E, E)]
                return acc
            acc = lax.fori_loop(0, H // 16, hb, jnp.zeros((E,), jnp.float32))
            out_v[pl.ds(t * E, E)] = acc
        pltpu.sync_copy(out_v, out_hbm.at[pl.ds(t0 * E, CH * E)])
        return carry

    lax.fori_loop(0, TPW // CH, chunk, 0)


def _sc_epilogue_block(lt_ref, weights_ref, topw_ref, tope_ref):
    probs, topw, tope = _epilogue(lt_ref[...])
    weights_ref[...] = probs
    topw_ref[...] = topw
    tope_ref[...] = tope


@jax.jit
def _router(hs2d, W):
    wt_tc, topw_tc, tope_tc = pl.pallas_call(
        _router_block,
        grid=(N_TC // BLK,),
        in_specs=[
            pl.BlockSpec((BLK, H), lambda i: (i, 0)),
            pl.BlockSpec((E, H), lambda i: (0, 0)),
        ],
        out_specs=[
            pl.BlockSpec((E, BLK), lambda i: (0, i)),
            pl.BlockSpec((TOPK, BLK), lambda i: (0, i)),
            pl.BlockSpec((TOPK, BLK), lambda i: (0, i)),
        ],
        out_shape=[
            jax.ShapeDtypeStruct((E, N_TC), jnp.float32),
            jax.ShapeDtypeStruct((TOPK, N_TC), jnp.float32),
            jax.ShapeDtypeStruct((TOPK, N_TC), jnp.int32),
        ],
    )(hs2d, W)

    sc_kernel = functools.partial(
        pl.kernel,
        mesh=plsc.VectorSubcoreMesh(core_axis_name="c", subcore_axis_name="s"),
        out_type=jax.ShapeDtypeStruct((N_SC * E,), jnp.float32),
        scratch_types=[
            pltpu.VMEM((H * E,), jnp.float32),
            pltpu.VMEM((CH, H), jnp.float32),
            pltpu.VMEM((CH * E,), jnp.float32),
        ],
    )(_sc_logits_body)
    wt_rounded = _bf16_round(W.T)
    lt_sc = sc_kernel(hs2d, wt_rounded.reshape(-1)).reshape(N_SC, E)

    wt_sc, topw_sc, tope_sc = pl.pallas_call(
        _sc_epilogue_block,
        grid=(1,),
        in_specs=[pl.BlockSpec((E, N_SC), lambda i: (0, 0))],
        out_specs=[
            pl.BlockSpec((E, N_SC), lambda i: (0, 0)),
            pl.BlockSpec((TOPK, N_SC), lambda i: (0, 0)),
            pl.BlockSpec((TOPK, N_SC), lambda i: (0, 0)),
        ],
        out_shape=[
            jax.ShapeDtypeStruct((E, N_SC), jnp.float32),
            jax.ShapeDtypeStruct((TOPK, N_SC), jnp.float32),
            jax.ShapeDtypeStruct((TOPK, N_SC), jnp.int32),
        ],
    )(lt_sc.T)

    weights = jnp.concatenate([wt_tc, wt_sc], axis=1).T
    top_weights = jnp.concatenate([topw_tc, topw_sc], axis=1).T
    top_experts = jnp.concatenate([tope_tc, tope_sc], axis=1).T
    return weights, top_weights, top_experts


def kernel(hidden_states, W):
    hs2d = hidden_states.reshape(-1, hidden_states.shape[-1])
    weights, top_weights, top_experts = _router(hs2d, W)
    weights = weights.astype(hidden_states.dtype)
    top_weights = top_weights.astype(hidden_states.dtype)
    return (weights, top_weights, top_experts)


# speedup vs baseline: 3.7684x; 1.4984x over previous
import functools

import jax
import jax.numpy as jnp
from jax.experimental import pallas as pl

E = 16
TOPK = 2
BLK = 1024


def _router_block(hs_ref, w_ref, weights_ref, topw_ref, tope_ref):
    hs = hs_ref[...]
    w = w_ref[...]
    lt = jax.lax.dot_general(
        w, hs, (((1,), (1,)), ((), ())), preferred_element_type=jnp.float32
    )
    m1 = jnp.max(lt, axis=0, keepdims=True)
    ex = jnp.exp(lt - m1)
    s = jnp.sum(ex, axis=0, keepdims=True)
    weights_ref[...] = ex * (1.0 / s)

    rows = jax.lax.broadcasted_iota(jnp.int32, lt.shape, 0)
    i1 = jnp.min(jnp.where(lt == m1, rows, E), axis=0, keepdims=True)
    masked = jnp.where(rows == i1, -jnp.inf, lt)
    m2 = jnp.max(masked, axis=0, keepdims=True)
    i2 = jnp.min(jnp.where(masked == m2, rows, E), axis=0, keepdims=True)

    e2 = jnp.exp(m2 - m1)
    tw1 = 1.0 / (1.0 + e2)
    topw_ref[...] = jnp.concatenate([tw1, e2 * tw1], axis=0)
    tope_ref[...] = jnp.concatenate([i1, i2], axis=0)


@functools.partial(jax.jit, static_argnames=("interpret",))
def _router(hs2d, W, interpret=False):
    n = hs2d.shape[0]
    h = hs2d.shape[1]
    grid = (n // BLK,)
    return pl.pallas_call(
        _router_block,
        grid=grid,
        in_specs=[
            pl.BlockSpec((BLK, h), lambda i: (i, 0)),
            pl.BlockSpec((E, h), lambda i: (0, 0)),
        ],
        out_specs=[
            pl.BlockSpec((E, BLK), lambda i: (0, i)),
            pl.BlockSpec((TOPK, BLK), lambda i: (0, i)),
            pl.BlockSpec((TOPK, BLK), lambda i: (0, i)),
        ],
        out_shape=[
            jax.ShapeDtypeStruct((E, n), jnp.float32),
            jax.ShapeDtypeStruct((TOPK, n), jnp.float32),
            jax.ShapeDtypeStruct((TOPK, n), jnp.int32),
        ],
        interpret=interpret,
    )(hs2d, W)


def kernel(hidden_states, W):
    hs2d = hidden_states.reshape(-1, hidden_states.shape[-1])
    weights_t, top_weights_t, top_experts_t = _router(hs2d, W)
    weights = weights_t.T.astype(hidden_states.dtype)
    top_weights = top_weights_t.T.astype(hidden_states.dtype)
    top_experts = top_experts_t.T
    return (weights, top_weights, top_experts)
